# baseline (speedup 1.0000x reference)
import functools

import jax
import jax.numpy as jnp
from jax import lax
from jax.experimental import pallas as pl
from jax.experimental.pallas import tpu as pltpu
from jax.experimental.pallas import tpu_sc as plsc

_ROWS = 1000000
_DIM = 64
_NC = 2
_HALF = _ROWS // _NC
_CH = 2000
_NCHC = _HALF // _CH
_NBUF = 2

_mesh = plsc.VectorSubcoreMesh(core_axis_name="c", subcore_axis_name="s")


@functools.partial(
    pl.kernel,
    out_type=jax.ShapeDtypeStruct((_ROWS, _DIM), jnp.float32),
    mesh=_mesh,
    compiler_params=pltpu.CompilerParams(needs_layout_passes=False),
    scratch_types=[
        pltpu.VMEM_SHARED((_NC, _NBUF, _CH, _DIM), jnp.float32),
        pltpu.VMEM((16,), jnp.int32),
        pltpu.VMEM((_DIM,), jnp.float32),
        pltpu.VMEM((8, _DIM), jnp.float32),
        pltpu.SemaphoreType.DMA,
        pltpu.SemaphoreType.DMA,
    ],
)
def _sc_body(
    emb_hbm, idx_hbm, new_hbm, out_hbm, bufs, idxv, newv, tilev, in_sem, out_sem
):
    cid = lax.axis_index("c")
    sid = lax.axis_index("s")
    base = pl.multiple_of(cid * _HALF, 8)
    pltpu.sync_copy(idx_hbm, idxv)
    pltpu.sync_copy(new_hbm, newv)
    idx = jnp.max(idxv[...])

    def in_cp(c, s):
        return pltpu.make_async_copy(
            emb_hbm.at[pl.ds(base + c * _CH, _CH), :], bufs.at[cid, s], in_sem
        )

    def out_cp(c, s):
        return pltpu.make_async_copy(
            bufs.at[cid, s], out_hbm.at[pl.ds(base + c * _CH, _CH), :], out_sem
        )

    @pl.when(sid == 0)
    def _():
        in_cp(0, 0).start()
        for c in range(_NCHC):
            s = c % _NBUF
            if c + 1 < _NCHC:
                s2 = (c + 1) % _NBUF
                if c + 1 >= _NBUF:
                    out_cp(c + 1 - _NBUF, s2).wait()
                in_cp(c + 1, s2).start()
            in_cp(c, s).wait()
            out_cp(c, s).start()
        for c in range(max(0, _NCHC - _NBUF), _NCHC):
            out_cp(c, c % _NBUF).wait()

    owns = (sid == 0) & (idx >= base) & (idx < base + _HALF)

    @pl.when(owns)
    def _():
        tile = pl.multiple_of((idx // 8) * 8, 8)
        local = idx - tile
        tin = pltpu.make_async_copy(emb_hbm.at[pl.ds(tile, 8), :], tilev, in_sem)
        tin.start()
        tin.wait()
        rows = jnp.full((16,), local, dtype=jnp.int32)
        for j in range(_DIM // 16):
            cols = lax.iota(jnp.int32, 16) + 16 * j
            plsc.store_scatter(tilev, [rows, cols], newv[pl.ds(16 * j, 16)])
        tout = pltpu.make_async_copy(tilev, out_hbm.at[pl.ds(tile, 8), :], out_sem)
        tout.start()
        tout.wait()


def kernel(embeddings, emb_index, new_emb):
    idx16 = jnp.full((16,), emb_index, dtype=jnp.int32)
    new_row = new_emb.reshape(_DIM)
    return _sc_body(embeddings, idx16, new_row)


# speedup vs baseline: 1.1462x; 1.1462x over previous
import jax
import jax.numpy as jnp
from jax.experimental import pallas as pl
from jax.experimental.pallas import tpu as pltpu

_ROWS = 1000000
_DIM = 64
_BLK = 20000
_GRID = _ROWS // _BLK


def _body(idx_ref, emb_ref, new_ref, out_ref):
    i = pl.program_id(0)
    out_ref[...] = emb_ref[...]
    idx = idx_ref[0]

    @pl.when(idx // _BLK == i)
    def _():
        out_ref[pl.ds(idx - i * _BLK, 1), :] = new_ref[...]


def kernel(embeddings, emb_index, new_emb):
    idx = jnp.asarray(emb_index, jnp.int32).reshape(1)
    new_row = new_emb.reshape(1, _DIM)
    grid_spec = pltpu.PrefetchScalarGridSpec(
        num_scalar_prefetch=1,
        grid=(_GRID,),
        in_specs=[
            pl.BlockSpec((_BLK, _DIM), lambda i, idx_ref: (i, 0)),
            pl.BlockSpec((1, _DIM), lambda i, idx_ref: (0, 0)),
        ],
        out_specs=pl.BlockSpec((_BLK, _DIM), lambda i, idx_ref: (i, 0)),
    )
    return pl.pallas_call(
        _body,
        grid_spec=grid_spec,
        out_shape=jax.ShapeDtypeStruct((_ROWS, _DIM), embeddings.dtype),
        compiler_params=pltpu.CompilerParams(needs_layout_passes=False),
    )(idx, embeddings, new_row)
